# baseline (speedup 1.0000x reference)
import functools

import jax
import jax.numpy as jnp
from jax import lax
from jax.experimental import pallas as pl
from jax.experimental.pallas import tpu as pltpu
from jax.experimental.pallas import tpu_sc as plsc

_B = 1024
_NB = 14
_LANES = 16
_NUM_CORES = 2
_NUM_SUBCORES = 16
_NUM_TECS = _NUM_CORES * _NUM_SUBCORES
_GROUP = 4


def _ram_layer_kernel(n_neurons, n_cols, xT, connp, memw, out,
                      conn_v, planes_v, mem_v, out_v, sem):
  n_per_tec = n_neurons // _NUM_TECS
  n_groups = n_per_tec // _GROUP
  cid = lax.axis_index("c")
  sid = lax.axis_index("s")
  wid = sid * _NUM_CORES + cid
  n0 = wid * n_per_tec

  pltpu.sync_copy(connp.at[pl.ds(n0, n_per_tec)], conn_v)

  def group_body(g, carry):
    gn0 = n0 + g * _GROUP
    descs = []
    for j in range(_GROUP):
      idx = conn_v.at[g * _GROUP + j]
      descs.append(
          pltpu.async_copy(xT.at[idx], planes_v.at[pl.ds(j * 16, 16)], sem))
    for j in range(_GROUP):
      pltpu.sync_copy(memw.at[gn0 + j], mem_v.at[pl.ds(j * 4096, 4096)])
    for d in descs:
      d.wait()

    for j in range(_GROUP):
      def vec_body(s, c, j=j):
        a = jnp.zeros((_LANES,), jnp.int32)
        for k in range(_NB):
          p = planes_v[j * 16 + k, pl.ds(s * _LANES, _LANES)]
          a = a + (p << k)
        widx = (a >> 2) + (j * 4096)
        w = plsc.load_gather(mem_v, [widx])
        bit = (w >> ((a & 3) << 3)) & 1
        out_v[j, pl.ds(s * _LANES, _LANES)] = bit
        return c
      lax.fori_loop(0, _B // _LANES, vec_body, 0)

    pltpu.sync_copy(out_v, out.at[pl.ds(gn0, _GROUP)])
    return carry

  lax.fori_loop(0, n_groups, group_body, 0)


def _ram_layer(xT, connp, memw, n_neurons):
  n_cols = xT.shape[0]
  n_per_tec = n_neurons // _NUM_TECS
  mesh = plsc.VectorSubcoreMesh(
      core_axis_name="c", subcore_axis_name="s",
      num_cores=_NUM_CORES, num_subcores=_NUM_SUBCORES)
  body = functools.partial(_ram_layer_kernel, n_neurons, n_cols)
  f = pl.kernel(
      body,
      out_type=jax.ShapeDtypeStruct((n_neurons, _B), jnp.int32),
      mesh=mesh,
      compiler_params=pltpu.CompilerParams(needs_layout_passes=False),
      scratch_types=[
          pltpu.VMEM((n_per_tec, 16), jnp.int32),
          pltpu.VMEM((_GROUP * 16, _B), jnp.int32),
          pltpu.VMEM((_GROUP * 4096,), jnp.int32),
          pltpu.VMEM((_GROUP, _B), jnp.int32),
          pltpu.SemaphoreType.DMA,
      ],
      name=f"ram_layer_n{n_neurons}",
  )
  return f(xT, connp, memw)


def _pack_words(mem_bool):
  n = mem_bool.shape[0]
  b = mem_bool.reshape(n, 4096, 4).astype(jnp.uint8)
  return lax.bitcast_convert_type(b, jnp.int32)


def kernel(input_bits, prev_state_bits, in_conn, in_mem, st_conn, st_mem):
  x = jnp.concatenate([input_bits, prev_state_bits], axis=1)
  xT = x.T.astype(jnp.int32)
  prevT = prev_state_bits.T.astype(jnp.int32)

  in_connp = jnp.pad(in_conn.astype(jnp.int32), ((0, 0), (0, 2)))
  st_connp = jnp.pad(st_conn.astype(jnp.int32), ((0, 0), (0, 2)))
  in_memw = _pack_words(in_mem)
  st_memw = _pack_words(st_mem)

  outT1 = _ram_layer(xT, in_connp, in_memw, in_conn.shape[0])
  yT = jnp.concatenate([outT1, prevT], axis=0)
  outT2 = _ram_layer(yT, st_connp, st_memw, st_conn.shape[0])

  input_out = outT1.T.astype(bool)
  next_state = outT2.T.astype(bool)
  return (input_out, next_state)


# speedup vs baseline: 2.1542x; 2.1542x over previous
import functools

import jax
import jax.numpy as jnp
from jax import lax
from jax.experimental import pallas as pl
from jax.experimental.pallas import tpu as pltpu
from jax.experimental.pallas import tpu_sc as plsc

_B = 1024
_NB = 14
_LANES = 16
_NUM_CORES = 2
_NUM_SUBCORES = 16
_NUM_TECS = _NUM_CORES * _NUM_SUBCORES
_GROUP = 8
_PW = 512
_PACK_BLK = 64


def _pack_body(m_ref, o_ref):
  m = m_ref[...].astype(jnp.int32).reshape(_PACK_BLK, 32, _PW)
  shifts = jnp.arange(32, dtype=jnp.int32)[None, :, None]
  o_ref[...] = jnp.sum(m << shifts, axis=1)


def _pack_bits(mem_bool):
  n = mem_bool.shape[0]
  return pl.pallas_call(
      _pack_body,
      grid=(n // _PACK_BLK,),
      in_specs=[pl.BlockSpec((_PACK_BLK, 32 * _PW), lambda i: (i, 0))],
      out_specs=pl.BlockSpec((_PACK_BLK, _PW), lambda i: (i, 0)),
      out_shape=jax.ShapeDtypeStruct((n, _PW), jnp.int32),
  )(mem_bool)


def _ram_layer_kernel(n_neurons, n_cols, xT, connp, memp, out,
                      conn_v, planes_v, mem_v, out_v, sem):
  n_per_tec = n_neurons // _NUM_TECS
  n_groups = n_per_tec // _GROUP
  cid = lax.axis_index("c")
  sid = lax.axis_index("s")
  wid = sid * _NUM_CORES + cid
  n0 = wid * n_per_tec

  pltpu.sync_copy(connp.at[pl.ds(n0, n_per_tec)], conn_v)
  iota16 = jnp.arange(_LANES, dtype=jnp.int32)

  def group_body(g, carry):
    gn0 = n0 + g * _GROUP
    descs = []
    for j in range(_GROUP):
      idx = conn_v.at[g * _GROUP + j]
      descs.append(
          pltpu.async_copy(xT.at[idx], planes_v.at[pl.ds(j * 16, 16)], sem))
    for j in range(_GROUP):
      pltpu.sync_copy(memp.at[gn0 + j], mem_v.at[pl.ds(j * _PW, _PW)])
    for d in descs:
      d.wait()

    for j in range(_GROUP):
      def vec_body(v, c, j=j):
        acc = [jnp.zeros((_LANES,), jnp.int32) for _ in range(4)]
        for k in range(_NB):
          pk = planes_v[j * 16 + k, pl.ds(v * _LANES, _LANES)]
          for t in range(4):
            if 8 * t >= k:
              term = (pk >> (8 * t - k)) & (1 << k)
            else:
              term = (pk << (k - 8 * t)) & (1 << k)
            acc[t] = acc[t] + term
        obase = j * (_B // 4) + v * _LANES + iota16
        for t in range(4):
          a = acc[t]
          w = plsc.load_gather(mem_v, [(a & (_PW - 1)) + j * _PW])
          bit = (w >> (a >> 9)) & 1
          if t == 0:
            plsc.store_scatter(out_v, [obase], bit)
          else:
            plsc.addupdate_scatter(out_v, [obase], bit << (8 * t))
        return c
      lax.fori_loop(0, _B // 64, vec_body, 0)

    for j in range(_GROUP):
      pltpu.sync_copy(out_v.at[pl.ds(j * (_B // 4), _B // 4)], out.at[gn0 + j])
    return carry

  lax.fori_loop(0, n_groups, group_body, 0)


def _ram_layer(xT, connp, memp, n_neurons):
  n_cols = xT.shape[0]
  n_per_tec = n_neurons // _NUM_TECS
  mesh = plsc.VectorSubcoreMesh(
      core_axis_name="c", subcore_axis_name="s",
      num_cores=_NUM_CORES, num_subcores=_NUM_SUBCORES)
  body = functools.partial(_ram_layer_kernel, n_neurons, n_cols)
  f = pl.kernel(
      body,
      out_type=jax.ShapeDtypeStruct((n_neurons, _B // 4), jnp.int32),
      mesh=mesh,
      compiler_params=pltpu.CompilerParams(needs_layout_passes=False),
      scratch_types=[
          pltpu.VMEM((n_per_tec, 16), jnp.int32),
          pltpu.VMEM((_GROUP * 16, _B // 4), jnp.int32),
          pltpu.VMEM((_GROUP * _PW,), jnp.int32),
          pltpu.VMEM((_GROUP * (_B // 4),), jnp.int32),
          pltpu.SemaphoreType.DMA,
      ],
      name=f"ram_layer_n{n_neurons}",
  )
  return f(xT, connp, memp)


def kernel(input_bits, prev_state_bits, in_conn, in_mem, st_conn, st_mem):
  def _to_planes(bits_T):
    t = bits_T.shape[0]
    return lax.bitcast_convert_type(
        bits_T.astype(jnp.int8).reshape(t, _B // 4, 4), jnp.int32)

  x = jnp.concatenate([input_bits, prev_state_bits], axis=1)
  xT = _to_planes(x.T)
  prevT = _to_planes(prev_state_bits.T)

  in_connp = jnp.pad(in_conn.astype(jnp.int32), ((0, 0), (0, 2)))
  st_connp = jnp.pad(st_conn.astype(jnp.int32), ((0, 0), (0, 2)))
  in_memp = _pack_bits(in_mem)
  st_memp = _pack_bits(st_mem)

  outT1 = _ram_layer(xT, in_connp, in_memp, in_conn.shape[0])
  yT = jnp.concatenate([outT1, prevT], axis=0)
  outT2 = _ram_layer(yT, st_connp, st_memp, st_conn.shape[0])

  def _from_planes(planes, n):
    b = lax.bitcast_convert_type(planes, jnp.int8).reshape(n, _B)
    return b.T.astype(bool)

  input_out = _from_planes(outT1, in_conn.shape[0])
  next_state = _from_planes(outT2, st_conn.shape[0])
  return (input_out, next_state)


# speedup vs baseline: 2.1642x; 1.0046x over previous
import functools

import jax
import jax.numpy as jnp
from jax import lax
from jax.experimental import pallas as pl
from jax.experimental.pallas import tpu as pltpu
from jax.experimental.pallas import tpu_sc as plsc

_B = 1024
_NB = 14
_LANES = 16
_NUM_CORES = 2
_NUM_SUBCORES = 16
_NUM_TECS = _NUM_CORES * _NUM_SUBCORES
_GROUP = 8
_PW = 512
_PACK_BLK = 64


def _pack_body(m_ref, o_ref):
  m = m_ref[...].astype(jnp.int32).reshape(_PACK_BLK, 32, _PW)
  shifts = jnp.arange(32, dtype=jnp.int32)[None, :, None]
  o_ref[...] = jnp.sum(m << shifts, axis=1)


def _pack_bits(mem_bool):
  n = mem_bool.shape[0]
  return pl.pallas_call(
      _pack_body,
      grid=(n // _PACK_BLK,),
      in_specs=[pl.BlockSpec((_PACK_BLK, 32 * _PW), lambda i: (i, 0))],
      out_specs=pl.BlockSpec((_PACK_BLK, _PW), lambda i: (i, 0)),
      out_shape=jax.ShapeDtypeStruct((n, _PW), jnp.int32),
  )(mem_bool)


def _ram_layer_kernel(n_neurons, n_cols, xT, connp, memp, out,
                      conn_v, planes_a, planes_b, mem_a, mem_b, out_v,
                      psem_a, psem_b, msem_a, msem_b):
  n_per_tec = n_neurons // _NUM_TECS
  n_groups = n_per_tec // _GROUP
  n_pairs = n_groups // 2
  cid = lax.axis_index("c")
  sid = lax.axis_index("s")
  wid = sid * _NUM_CORES + cid
  n0 = wid * n_per_tec

  pltpu.sync_copy(connp.at[pl.ds(n0, n_per_tec)], conn_v)
  iota16 = jnp.arange(_LANES, dtype=jnp.int32)

  def fire(g, planes, mem, psem, msem):
    gn0 = n0 + g * _GROUP
    for j in range(_GROUP):
      idx = conn_v.at[g * _GROUP + j]
      pltpu.async_copy(xT.at[idx], planes.at[pl.ds(j * 16, 16)], psem)
    pltpu.async_copy(memp.at[pl.ds(gn0, _GROUP)], mem, msem)

  def drain(planes, mem, psem, msem):
    pltpu.make_async_copy(xT.at[pl.ds(0, _GROUP * 16)], planes, psem).wait()
    pltpu.make_async_copy(memp.at[pl.ds(0, _GROUP)], mem, msem).wait()

  def compute(g, planes, mem):
    gn0 = n0 + g * _GROUP
    for j in range(_GROUP):
      jrow = jnp.full((_LANES,), j, dtype=jnp.int32)
      def vec_body(v, c, j=j, jrow=jrow):
        ps = [planes[j * 16 + k, pl.ds(v * _LANES, _LANES)]
              for k in range(_NB)]
        quads = []
        for k0 in range(0, 12, 4):
          q = (ps[k0] + (ps[k0 + 1] << 1)
               + (ps[k0 + 2] << 2) + (ps[k0 + 3] << 3))
          quads.append((k0, q, 0xF))
        quads.append((12, ps[12] + (ps[13] << 1), 0x3))
        obase = v * _LANES + iota16
        for t in range(4):
          a = None
          for k0, q, m in quads:
            s = 8 * t - k0
            if s >= 0:
              term = (q >> s) & (m << k0)
            else:
              term = (q << (-s)) & (m << k0)
            a = term if a is None else a + term
          w = plsc.load_gather(mem, [jrow, a & (_PW - 1)])
          bit = (w >> (a >> 9)) & 1
          if t == 0:
            plsc.store_scatter(out_v, [jrow, obase], bit)
          else:
            plsc.addupdate_scatter(out_v, [jrow, obase], bit << (8 * t))
        return c
      lax.fori_loop(0, _B // 64, vec_body, 0)
    pltpu.sync_copy(out_v, out.at[pl.ds(gn0, _GROUP)])

  fire(0, planes_a, mem_a, psem_a, msem_a)

  def pair_body(p, c):
    g0 = 2 * p
    fire(g0 + 1, planes_b, mem_b, psem_b, msem_b)
    drain(planes_a, mem_a, psem_a, msem_a)
    compute(g0, planes_a, mem_a)

    @pl.when(p + 1 < n_pairs)
    def _prefetch():
      fire(g0 + 2, planes_a, mem_a, psem_a, msem_a)

    drain(planes_b, mem_b, psem_b, msem_b)
    compute(g0 + 1, planes_b, mem_b)
    return c

  lax.fori_loop(0, n_pairs, pair_body, 0)


def _ram_layer(xT, connp, memp, n_neurons):
  n_cols = xT.shape[0]
  n_per_tec = n_neurons // _NUM_TECS
  mesh = plsc.VectorSubcoreMesh(
      core_axis_name="c", subcore_axis_name="s",
      num_cores=_NUM_CORES, num_subcores=_NUM_SUBCORES)
  body = functools.partial(_ram_layer_kernel, n_neurons, n_cols)
  f = pl.kernel(
      body,
      out_type=jax.ShapeDtypeStruct((n_neurons, _B // 4), jnp.int32),
      mesh=mesh,
      compiler_params=pltpu.CompilerParams(needs_layout_passes=False),
      scratch_types=[
          pltpu.VMEM((n_per_tec, 16), jnp.int32),
          pltpu.VMEM((_GROUP * 16, _B // 4), jnp.int32),
          pltpu.VMEM((_GROUP * 16, _B // 4), jnp.int32),
          pltpu.VMEM((_GROUP, _PW), jnp.int32),
          pltpu.VMEM((_GROUP, _PW), jnp.int32),
          pltpu.VMEM((_GROUP, _B // 4), jnp.int32),
          pltpu.SemaphoreType.DMA,
          pltpu.SemaphoreType.DMA,
          pltpu.SemaphoreType.DMA,
          pltpu.SemaphoreType.DMA,
      ],
      name=f"ram_layer_n{n_neurons}",
  )
  return f(xT, connp, memp)


def kernel(input_bits, prev_state_bits, in_conn, in_mem, st_conn, st_mem):
  def _to_planes(bits_T):
    t = bits_T.shape[0]
    return lax.bitcast_convert_type(
        bits_T.astype(jnp.int8).reshape(t, _B // 4, 4), jnp.int32)

  x = jnp.concatenate([input_bits, prev_state_bits], axis=1)
  xT = _to_planes(x.T)
  prevT = _to_planes(prev_state_bits.T)

  in_connp = jnp.pad(in_conn.astype(jnp.int32), ((0, 0), (0, 2)))
  st_connp = jnp.pad(st_conn.astype(jnp.int32), ((0, 0), (0, 2)))
  in_memp = _pack_bits(in_mem)
  st_memp = _pack_bits(st_mem)

  outT1 = _ram_layer(xT, in_connp, in_memp, in_conn.shape[0])
  yT = jnp.concatenate([outT1, prevT], axis=0)
  outT2 = _ram_layer(yT, st_connp, st_memp, st_conn.shape[0])

  def _from_planes(planes, n):
    b = lax.bitcast_convert_type(planes, jnp.int8).reshape(n, _B)
    return b.T.astype(bool)

  input_out = _from_planes(outT1, in_conn.shape[0])
  next_state = _from_planes(outT2, st_conn.shape[0])
  return (input_out, next_state)


# speedup vs baseline: 2.1784x; 1.0066x over previous
import functools

import jax
import jax.numpy as jnp
from jax import lax
from jax.experimental import pallas as pl
from jax.experimental.pallas import tpu as pltpu
from jax.experimental.pallas import tpu_sc as plsc

_B = 1024
_NB = 14
_LANES = 16
_NUM_CORES = 2
_NUM_SUBCORES = 16
_NUM_TECS = _NUM_CORES * _NUM_SUBCORES
_GROUP = 8
_PW = 512
_PACK_BLK = 64


def _pack_body(m_ref, o_ref):
  m = m_ref[...].astype(jnp.int32).reshape(_PACK_BLK, 32, _PW)
  shifts = jnp.arange(32, dtype=jnp.int32)[None, :, None]
  o_ref[...] = jnp.sum(m << shifts, axis=1)


def _pack_bits(mem_bool):
  n = mem_bool.shape[0]
  return pl.pallas_call(
      _pack_body,
      grid=(n // _PACK_BLK,),
      in_specs=[pl.BlockSpec((_PACK_BLK, 32 * _PW), lambda i: (i, 0))],
      out_specs=pl.BlockSpec((_PACK_BLK, _PW), lambda i: (i, 0)),
      out_shape=jax.ShapeDtypeStruct((n, _PW), jnp.int32),
  )(mem_bool)


def _ram_layer_kernel(n_neurons, n_cols, xT, connp, memp, out,
                      conn_v, planes_a, planes_b, mem_a, mem_b, out_v,
                      psem_a, psem_b, msem_a, msem_b):
  n_per_tec = n_neurons // _NUM_TECS
  n_groups = n_per_tec // _GROUP
  n_pairs = n_groups // 2
  cid = lax.axis_index("c")
  sid = lax.axis_index("s")
  wid = sid * _NUM_CORES + cid
  n0 = wid * n_per_tec

  pltpu.sync_copy(connp.at[pl.ds(n0, n_per_tec)], conn_v)
  iota16 = jnp.arange(_LANES, dtype=jnp.int32)

  def fire(g, planes, mem, psem, msem):
    gn0 = n0 + g * _GROUP
    for j in range(_GROUP):
      idx = conn_v.at[g * _GROUP + j]
      pltpu.async_copy(xT.at[idx], planes.at[pl.ds(j * 16, 16)], psem)
    pltpu.async_copy(memp.at[pl.ds(gn0, _GROUP)], mem, msem)

  def drain(planes, mem, psem, msem):
    pltpu.make_async_copy(xT.at[pl.ds(0, _GROUP * 16)], planes, psem).wait()
    pltpu.make_async_copy(memp.at[pl.ds(0, _GROUP)], mem, msem).wait()

  jrows = [jnp.full((_LANES,), j, dtype=jnp.int32) for j in range(_GROUP)]

  def compute(g, planes, mem):
    gn0 = n0 + g * _GROUP

    def vec_body(v, c):
      obase = v * _LANES + iota16
      for j in range(_GROUP):
        ps = [planes[j * 16 + k, pl.ds(v * _LANES, _LANES)]
              for k in range(_NB)]
        quads = []
        for k0 in range(0, 12, 4):
          q = (ps[k0] + (ps[k0 + 1] << 1)
               + (ps[k0 + 2] << 2) + (ps[k0 + 3] << 3))
          quads.append((k0, q, 0xF))
        quads.append((12, ps[12] + (ps[13] << 1), 0x3))
        for t in range(4):
          a = None
          for k0, q, m in quads:
            s = 8 * t - k0
            if s >= 0:
              term = (q >> s) & (m << k0)
            else:
              term = (q << (-s)) & (m << k0)
            a = term if a is None else a + term
          w = plsc.load_gather(mem, [jrows[j], a & (_PW - 1)])
          bit = (w >> (a >> 9)) & 1
          if t == 0:
            plsc.store_scatter(out_v, [jrows[j], obase], bit)
          else:
            plsc.addupdate_scatter(out_v, [jrows[j], obase], bit << (8 * t))
      return c

    lax.fori_loop(0, _B // 64, vec_body, 0)
    pltpu.sync_copy(out_v, out.at[pl.ds(gn0, _GROUP)])

  fire(0, planes_a, mem_a, psem_a, msem_a)

  def pair_body(p, c):
    g0 = 2 * p
    fire(g0 + 1, planes_b, mem_b, psem_b, msem_b)
    drain(planes_a, mem_a, psem_a, msem_a)
    compute(g0, planes_a, mem_a)

    @pl.when(p + 1 < n_pairs)
    def _prefetch():
      fire(g0 + 2, planes_a, mem_a, psem_a, msem_a)

    drain(planes_b, mem_b, psem_b, msem_b)
    compute(g0 + 1, planes_b, mem_b)
    return c

  lax.fori_loop(0, n_pairs, pair_body, 0)


def _ram_layer(xT, connp, memp, n_neurons):
  n_cols = xT.shape[0]
  n_per_tec = n_neurons // _NUM_TECS
  mesh = plsc.VectorSubcoreMesh(
      core_axis_name="c", subcore_axis_name="s",
      num_cores=_NUM_CORES, num_subcores=_NUM_SUBCORES)
  body = functools.partial(_ram_layer_kernel, n_neurons, n_cols)
  f = pl.kernel(
      body,
      out_type=jax.ShapeDtypeStruct((n_neurons, _B // 4), jnp.int32),
      mesh=mesh,
      compiler_params=pltpu.CompilerParams(needs_layout_passes=False),
      scratch_types=[
          pltpu.VMEM((n_per_tec, 16), jnp.int32),
          pltpu.VMEM((_GROUP * 16, _B // 4), jnp.int32),
          pltpu.VMEM((_GROUP * 16, _B // 4), jnp.int32),
          pltpu.VMEM((_GROUP, _PW), jnp.int32),
          pltpu.VMEM((_GROUP, _PW), jnp.int32),
          pltpu.VMEM((_GROUP, _B // 4), jnp.int32),
          pltpu.SemaphoreType.DMA,
          pltpu.SemaphoreType.DMA,
          pltpu.SemaphoreType.DMA,
          pltpu.SemaphoreType.DMA,
      ],
      name=f"ram_layer_n{n_neurons}",
  )
  return f(xT, connp, memp)


def kernel(input_bits, prev_state_bits, in_conn, in_mem, st_conn, st_mem):
  def _to_planes(bits_T):
    t = bits_T.shape[0]
    return lax.bitcast_convert_type(
        bits_T.astype(jnp.int8).reshape(t, _B // 4, 4), jnp.int32)

  x = jnp.concatenate([input_bits, prev_state_bits], axis=1)
  xT = _to_planes(x.T)
  prevT = _to_planes(prev_state_bits.T)

  in_connp = jnp.pad(in_conn.astype(jnp.int32), ((0, 0), (0, 2)))
  st_connp = jnp.pad(st_conn.astype(jnp.int32), ((0, 0), (0, 2)))
  in_memp = _pack_bits(in_mem)
  st_memp = _pack_bits(st_mem)

  outT1 = _ram_layer(xT, in_connp, in_memp, in_conn.shape[0])
  yT = jnp.concatenate([outT1, prevT], axis=0)
  outT2 = _ram_layer(yT, st_connp, st_memp, st_conn.shape[0])

  def _from_planes(planes, n):
    b = lax.bitcast_convert_type(planes, jnp.int8).reshape(n, _B)
    return b.T.astype(bool)

  input_out = _from_planes(outT1, in_conn.shape[0])
  next_state = _from_planes(outT2, st_conn.shape[0])
  return (input_out, next_state)
